# Initial kernel scaffold; baseline (speedup 1.0000x reference)
import jax
import jax.numpy as jnp
from jax.experimental import pallas as pl


def kernel(inputs, b, e):
    raise NotImplementedError("write your pallas kernel here")


# speedup vs baseline: 1.1772x; 1.1772x over previous
import functools

import jax
import jax.numpy as jnp
from jax import lax
from jax.experimental import pallas as pl
from jax.experimental.pallas import tpu as pltpu
from jax.experimental.pallas import tpu_sc as plsc

V = 32768
D = 1024
B = 65536
NC = 2
NS = 16
NW = NC * NS
ROWS_PER_W = B // NW
C = 32
NCHUNK = ROWS_PER_W // C
LANES = 16


def _sc_body(table_hbm, b_hbm, e_hbm, out_hbm,
             idx_b, idx_e, buf_b, buf_e, sem_b, sem_e):
    wid = lax.axis_index("s") * NC + lax.axis_index("c")
    base = wid * ROWS_PER_W
    pltpu.sync_copy(b_hbm.at[pl.ds(base, ROWS_PER_W)], idx_b)
    pltpu.sync_copy(e_hbm.at[pl.ds(base, ROWS_PER_W)], idx_e)

    def chunk_body(cix, carry):
        off = cix * C
        cb = pltpu.async_copy(table_hbm.at[idx_b.at[pl.ds(off, C)]], buf_b, sem_b)
        ce = pltpu.async_copy(table_hbm.at[idx_e.at[pl.ds(off, C)]], buf_e, sem_e)
        cb.wait()
        ce.wait()

        def row_body(i, rcarry):
            for j in range(D // LANES):
                s = pl.ds(j * LANES, LANES)
                buf_b[i, s] = buf_b[i, s] + buf_e[i, s]
            return rcarry
        lax.fori_loop(0, C, row_body, 0, unroll=False)

        pltpu.sync_copy(buf_b, out_hbm.at[pl.ds(base + off, C)])
        return carry

    lax.fori_loop(0, NCHUNK, chunk_body, 0, unroll=False)


_mesh = plsc.VectorSubcoreMesh(core_axis_name="c", subcore_axis_name="s")

_span_call = functools.partial(
    pl.kernel,
    out_type=jax.ShapeDtypeStruct((B, D), jnp.float32),
    mesh=_mesh,
    scratch_types=[
        pltpu.VMEM((ROWS_PER_W,), jnp.int32),
        pltpu.VMEM((ROWS_PER_W,), jnp.int32),
        pltpu.VMEM((C, D), jnp.float32),
        pltpu.VMEM((C, D), jnp.float32),
        pltpu.SemaphoreType.DMA,
        pltpu.SemaphoreType.DMA,
    ],
)(_sc_body)


def kernel(inputs, b, e):
    return _span_call(inputs, b.astype(jnp.int32), e.astype(jnp.int32))


# speedup vs baseline: 2.1360x; 1.8146x over previous
import functools

import jax
import jax.numpy as jnp
from jax import lax
from jax.experimental import pallas as pl
from jax.experimental.pallas import tpu as pltpu
from jax.experimental.pallas import tpu_sc as plsc

V = 32768
D = 1024
B = 65536
NC = 2
NS = 16
NW = NC * NS
ROWS_PER_W = B // NW
C = 16
NCHUNK = ROWS_PER_W // C
NSTAGE = 2
LANES = 16


def _sc_body(table_hbm, b_hbm, e_hbm, out_hbm, idx_b, idx_e,
             bb0, bb1, be0, be1, bo0, bo1,
             semb0, semb1, seme0, seme1, semo0, semo1):
    buf_b = (bb0, bb1)
    buf_e = (be0, be1)
    buf_o = (bo0, bo1)
    sem_b = (semb0, semb1)
    sem_e = (seme0, seme1)
    sem_o = (semo0, semo1)

    wid = lax.axis_index("s") * NC + lax.axis_index("c")
    base = wid * ROWS_PER_W
    pltpu.sync_copy(b_hbm.at[pl.ds(base, ROWS_PER_W)], idx_b)
    pltpu.sync_copy(e_hbm.at[pl.ds(base, ROWS_PER_W)], idx_e)

    def issue_gathers(cix, p):
        off = cix * C
        pltpu.async_copy(table_hbm.at[idx_b.at[pl.ds(off, C)]], buf_b[p],
                         sem_b[p])
        pltpu.async_copy(table_hbm.at[idx_e.at[pl.ds(off, C)]], buf_e[p],
                         sem_e[p])

    for p in range(NSTAGE):
        issue_gathers(p, p)

    def round_body(g, carry):
        for p in range(NSTAGE):
            cix = g * NSTAGE + p
            pltpu.make_async_copy(table_hbm.at[idx_b.at[pl.ds(0, C)]],
                                  buf_b[p], sem_b[p]).wait()
            pltpu.make_async_copy(table_hbm.at[idx_e.at[pl.ds(0, C)]],
                                  buf_e[p], sem_e[p]).wait()

            def row_body(i, rcarry, p=p):
                for j in range(D // LANES):
                    s = pl.ds(j * LANES, LANES)
                    buf_o[p][i, s] = buf_b[p][i, s] + buf_e[p][i, s]
                return rcarry
            lax.fori_loop(0, C, row_body, 0, unroll=False)

            @pl.when(cix + NSTAGE < NCHUNK)
            def _(cix=cix, p=p):
                issue_gathers(cix + NSTAGE, p)

            @pl.when(cix >= NSTAGE)
            def _(p=p):
                pltpu.make_async_copy(buf_o[p], out_hbm.at[pl.ds(base, C)],
                                      sem_o[p]).wait()

            pltpu.async_copy(buf_o[p], out_hbm.at[pl.ds(base + cix * C, C)],
                             sem_o[p])
        return carry

    lax.fori_loop(0, NCHUNK // NSTAGE, round_body, 0, unroll=False)

    for p in range(NSTAGE):
        pltpu.make_async_copy(buf_o[p], out_hbm.at[pl.ds(base, C)],
                              sem_o[p]).wait()


_mesh = plsc.VectorSubcoreMesh(core_axis_name="c", subcore_axis_name="s")

_span_call = functools.partial(
    pl.kernel,
    out_type=jax.ShapeDtypeStruct((B, D), jnp.float32),
    mesh=_mesh,
    scratch_types=[
        pltpu.VMEM((ROWS_PER_W,), jnp.int32),
        pltpu.VMEM((ROWS_PER_W,), jnp.int32),
    ] + [pltpu.VMEM((C, D), jnp.float32)] * 6
      + [pltpu.SemaphoreType.DMA] * 6,
)(_sc_body)


def kernel(inputs, b, e):
    return _span_call(inputs, b.astype(jnp.int32), e.astype(jnp.int32))


# speedup vs baseline: 2.1890x; 1.0248x over previous
import functools

import jax
import jax.numpy as jnp
from jax import lax
from jax.experimental import pallas as pl
from jax.experimental.pallas import tpu as pltpu
from jax.experimental.pallas import tpu_sc as plsc

V = 32768
D = 1024
B = 65536
NC = 2
NS = 16
NW = NC * NS
ROWS_PER_W = B // NW
C = 8
NCHUNK = ROWS_PER_W // C
NSTAGE = 4
LANES = 16


def _sc_body(table_hbm, b_hbm, e_hbm, out_hbm, idx_b, idx_e,
             bo0, bo1, bo2, bo3, be0, be1, be2, be3,
             smb0, smb1, smb2, smb3, sme0, sme1, sme2, sme3,
             smo0, smo1, smo2, smo3):
    buf_o = (bo0, bo1, bo2, bo3)
    buf_e = (be0, be1, be2, be3)
    sem_b = (smb0, smb1, smb2, smb3)
    sem_e = (sme0, sme1, sme2, sme3)
    sem_o = (smo0, smo1, smo2, smo3)

    wid = lax.axis_index("s") * NC + lax.axis_index("c")
    base = wid * ROWS_PER_W
    pltpu.sync_copy(b_hbm.at[pl.ds(base, ROWS_PER_W)], idx_b)
    pltpu.sync_copy(e_hbm.at[pl.ds(base, ROWS_PER_W)], idx_e)

    def issue_gathers(cix, p):
        off = cix * C
        pltpu.async_copy(table_hbm.at[idx_b.at[pl.ds(off, C)]], buf_o[p],
                         sem_b[p])
        pltpu.async_copy(table_hbm.at[idx_e.at[pl.ds(off, C)]], buf_e[p],
                         sem_e[p])

    def wait_out(p):
        pltpu.make_async_copy(buf_o[p], out_hbm.at[pl.ds(base, C)],
                              sem_o[p]).wait()

    for p in range(2):
        issue_gathers(p, p)

    def round_body(g, carry):
        for p in range(NSTAGE):
            cix = g * NSTAGE + p
            pf = (p + 2) % NSTAGE

            @pl.when(cix >= 2)
            def _(pf=pf):
                wait_out(pf)

            @pl.when(cix + 2 < NCHUNK)
            def _(cix=cix, pf=pf):
                issue_gathers(cix + 2, pf)

            pltpu.make_async_copy(table_hbm.at[idx_b.at[pl.ds(0, C)]],
                                  buf_o[p], sem_b[p]).wait()
            pltpu.make_async_copy(table_hbm.at[idx_e.at[pl.ds(0, C)]],
                                  buf_e[p], sem_e[p]).wait()

            def row_body(i, rcarry, p=p):
                for j in range(D // LANES):
                    s = pl.ds(j * LANES, LANES)
                    plsc.addupdate(buf_o[p].at[i, s], buf_e[p][i, s])
                return rcarry
            lax.fori_loop(0, C, row_body, 0, unroll=False)

            pltpu.async_copy(buf_o[p], out_hbm.at[pl.ds(base + cix * C, C)],
                             sem_o[p])
        return carry

    lax.fori_loop(0, NCHUNK // NSTAGE, round_body, 0, unroll=False)

    for cix in (NCHUNK - 2, NCHUNK - 1):
        wait_out(cix % NSTAGE)


_mesh = plsc.VectorSubcoreMesh(core_axis_name="c", subcore_axis_name="s")

_span_call = functools.partial(
    pl.kernel,
    out_type=jax.ShapeDtypeStruct((B, D), jnp.float32),
    mesh=_mesh,
    scratch_types=[
        pltpu.VMEM((ROWS_PER_W,), jnp.int32),
        pltpu.VMEM((ROWS_PER_W,), jnp.int32),
    ] + [pltpu.VMEM((C, D), jnp.float32)] * (2 * NSTAGE)
      + [pltpu.SemaphoreType.DMA] * (3 * NSTAGE),
)(_sc_body)


def kernel(inputs, b, e):
    return _span_call(inputs, b.astype(jnp.int32), e.astype(jnp.int32))
